# baseline (speedup 1.0000x reference)
import functools

import jax
import jax.numpy as jnp
from jax import lax
from jax.experimental import pallas as pl
from jax.experimental.pallas import tpu as pltpu
from jax.experimental.pallas import tpu_sc as plsc


def _make_gather(n_rows: int, seq: int, emb_dim: int):
    info = plsc.get_sparse_core_info()
    nw = info.num_cores * info.num_subcores
    lb_n = seq // 8
    bb_n = n_rows // 128
    n_tiles = lb_n * bb_n
    tiles_per_w = n_tiles // nw
    jobs_per_w = tiles_per_w * 8
    eb_n = emb_dim // 8

    mesh = plsc.VectorSubcoreMesh(core_axis_name="c", subcore_axis_name="s")

    @functools.partial(
        pl.kernel,
        mesh=mesh,
        out_type=jax.ShapeDtypeStruct((seq * eb_n * bb_n, 8 * 128), jnp.float32),
        scratch_types=[
            pltpu.VMEM((2, 8, 128), jnp.int32),
            pltpu.VMEM((2, 128, emb_dim), jnp.float32),
            pltpu.VMEM((2, emb_dim // 8, 8 * 128), jnp.float32),
            pltpu.SemaphoreType.DMA,
            pltpu.SemaphoreType.DMA,
            pltpu.SemaphoreType.DMA,
        ],
        compiler_params=pltpu.CompilerParams(
            use_tc_tiling_on_sc=False, needs_layout_passes=False
        ),
    )
    def gather(table_hbm, idx_hbm, out_hbm, idx_v, rows_v, pan_v, isem, gsem, osem):
        wid = lax.axis_index("s") * info.num_cores + lax.axis_index("c")
        t0 = wid * tiles_per_w

        def idx_copy(rt):
            t = t0 + rt
            return pltpu.make_async_copy(
                idx_hbm.at[t // bb_n].at[t % bb_n], idx_v.at[rt % 2], isem
            )

        def gather_copy(rt, lin, s):
            return pltpu.make_async_copy(
                table_hbm.at[idx_v.at[rt % 2].at[lin]], rows_v.at[s], gsem
            )

        def out_copy(j, eblk, s):
            t = t0 + j // 8
            row = ((t // bb_n) * 8 + (j % 8)) * (eb_n * bb_n) + eblk * bb_n + (t % bb_n)
            return pltpu.make_async_copy(
                pan_v.at[s].at[eblk], out_hbm.at[row], osem
            )

        jiota = [jnp.arange(16, dtype=jnp.int32) + (jb * 16) for jb in range(8)]

        idx_copy(0).start()
        idx_copy(0).wait()
        if tiles_per_w > 1:
            idx_copy(1).start()
        gather_copy(0, 0, 0).start()

        def body(i, carry):
            for s in (0, 1):
                j = 2 * i + s
                gather_copy(0, 0, s).wait()

                @pl.when(j + 1 < jobs_per_w)
                def _():
                    rt1 = (j + 1) // 8
                    lin1 = (j + 1) % 8

                    @pl.when(lin1 == 0)
                    def _():
                        idx_copy(0).wait()

                        @pl.when(rt1 + 1 < tiles_per_w)
                        def _():
                            idx_copy(rt1 + 1).start()

                    gather_copy(rt1, lin1, 1 - s).start()

                @pl.when(j >= 2)
                def _():
                    for eblk in range(eb_n):
                        out_copy(0, eblk, s).wait()

                for e in range(emb_dim):
                    col = jnp.full((16,), e, dtype=jnp.int32)
                    for jb in range(8):
                        v = plsc.load_gather(rows_v.at[s], [jiota[jb], col])
                        pan_v[s, e // 8, pl.ds((e % 8) * 128 + jb * 16, 16)] = v

                for eblk in range(eb_n):
                    out_copy(j, eblk, s).start()
            return carry

        lax.fori_loop(0, jobs_per_w // 2, body, 0)
        for s in (0, 1):
            for eblk in range(eb_n):
                out_copy(0, eblk, s).wait()

    return gather


def kernel(x, embedding_matrix):
    b, l = x.shape
    n_emb, emb_dim = embedding_matrix.shape
    xp = x.reshape(b // 128, 128, l // 8, 8).transpose(2, 0, 3, 1)
    out = _make_gather(b, l, emb_dim)(embedding_matrix, xp)
    out5 = out.reshape(l, emb_dim // 8, b // 128, 8, 128)
    return out5.transpose(2, 4, 0, 1, 3).reshape(b, l, emb_dim)


# speedup vs baseline: 1.5307x; 1.5307x over previous
import functools

import jax
import jax.numpy as jnp
from jax import lax
from jax.experimental import pallas as pl
from jax.experimental.pallas import tpu as pltpu
from jax.experimental.pallas import tpu_sc as plsc


def _make_gather(n_rows: int, seq: int, emb_dim: int):
    info = plsc.get_sparse_core_info()
    nw = info.num_cores * info.num_subcores
    lb_n = seq // 8
    bb_n = n_rows // 128
    n_tiles = lb_n * bb_n
    tiles_per_w = n_tiles // nw
    jobs_per_w = tiles_per_w * 8
    eb_n = emb_dim // 8

    mesh = plsc.VectorSubcoreMesh(core_axis_name="c", subcore_axis_name="s")

    @functools.partial(
        pl.kernel,
        mesh=mesh,
        out_type=jax.ShapeDtypeStruct((seq * eb_n * bb_n, 8 * 128), jnp.float32),
        scratch_types=[
            pltpu.VMEM((2, 8, 128), jnp.int32),
            pltpu.VMEM((128, emb_dim), jnp.float32),
            pltpu.VMEM((128, emb_dim), jnp.float32),
            pltpu.VMEM((128 * emb_dim,), jnp.float32),
            pltpu.VMEM((128 * emb_dim,), jnp.float32),
            pltpu.SemaphoreType.DMA,
            pltpu.SemaphoreType.DMA,
            pltpu.SemaphoreType.DMA,
        ],
        compiler_params=pltpu.CompilerParams(
            use_tc_tiling_on_sc=False, needs_layout_passes=False
        ),
    )
    def gather(
        table_hbm, idx_hbm, out_hbm, idx_v, rows0, rows1, pan0, pan1, isem, gsem, osem
    ):
        rows = (rows0, rows1)
        pans = (pan0, pan1)
        wid = lax.axis_index("s") * info.num_cores + lax.axis_index("c")
        t0 = wid * tiles_per_w

        def idx_copy(rt):
            t = t0 + rt
            return pltpu.make_async_copy(
                idx_hbm.at[t // bb_n].at[t % bb_n], idx_v.at[rt % 2], isem
            )

        def gather_copy(rt, lin, s):
            return pltpu.make_async_copy(
                table_hbm.at[idx_v.at[rt % 2].at[lin]], rows[s], gsem
            )

        def out_copy(j, eblk, s):
            t = t0 + j // 8
            row = ((t // bb_n) * 8 + (j % 8)) * (eb_n * bb_n) + eblk * bb_n + (t % bb_n)
            return pltpu.make_async_copy(
                pans[s].at[pl.ds(eblk * 1024, 1024)], out_hbm.at[row], osem
            )

        iota16 = jnp.arange(16, dtype=jnp.int32)
        jiota = [iota16 + (jb * 16) for jb in range(8)]

        idx_copy(0).start()
        idx_copy(0).wait()
        if tiles_per_w > 1:
            idx_copy(1).start()
        gather_copy(0, 0, 0).start()

        def body(i, carry):
            for s in (0, 1):
                j = 2 * i + s
                gather_copy(0, 0, s).wait()

                @pl.when(j + 1 < jobs_per_w)
                def _():
                    rt1 = (j + 1) // 8
                    lin1 = (j + 1) % 8

                    @pl.when(lin1 == 0)
                    def _():
                        idx_copy(0).wait()

                        @pl.when(rt1 + 1 < tiles_per_w)
                        def _():
                            idx_copy(rt1 + 1).start()

                    gather_copy(rt1, lin1, 1 - s).start()

                @pl.when(j >= 2)
                def _():
                    for eblk in range(eb_n):
                        out_copy(0, eblk, s).wait()

                for e0 in range(emb_dim):
                    colv = (iota16 + e0) & (emb_dim - 1)
                    colsh = colv << 7
                    for jb in range(8):
                        v = plsc.load_gather(rows[s], [jiota[jb], colv])
                        plsc.store_scatter(pans[s], [colsh + jiota[jb]], v)

                for eblk in range(eb_n):
                    out_copy(j, eblk, s).start()
            return carry

        lax.fori_loop(0, jobs_per_w // 2, body, 0)
        for s in (0, 1):
            for eblk in range(eb_n):
                out_copy(0, eblk, s).wait()

    return gather


def kernel(x, embedding_matrix):
    b, l = x.shape
    n_emb, emb_dim = embedding_matrix.shape
    xp = x.reshape(b // 128, 128, l // 8, 8).transpose(2, 0, 3, 1)
    out = _make_gather(b, l, emb_dim)(embedding_matrix, xp)
    out5 = out.reshape(l, emb_dim // 8, b // 128, 8, 128)
    return out5.transpose(2, 4, 0, 1, 3).reshape(b, l, emb_dim)


# speedup vs baseline: 1.5325x; 1.0012x over previous
import functools

import jax
import jax.numpy as jnp
from jax import lax
from jax.experimental import pallas as pl
from jax.experimental.pallas import tpu as pltpu
from jax.experimental.pallas import tpu_sc as plsc


def _make_gather(n_rows: int, seq: int, emb_dim: int):
    info = plsc.get_sparse_core_info()
    nw = info.num_cores * info.num_subcores
    lb_n = seq // 8
    bb_n = n_rows // 128
    n_tiles = lb_n * bb_n
    tiles_per_w = n_tiles // nw
    jobs_per_w = tiles_per_w * 8
    eb_n = emb_dim // 8

    mesh = plsc.VectorSubcoreMesh(core_axis_name="c", subcore_axis_name="s")

    @functools.partial(
        pl.kernel,
        mesh=mesh,
        out_type=jax.ShapeDtypeStruct((seq * eb_n * bb_n, 8 * 128), jnp.float32),
        scratch_types=[
            pltpu.VMEM((2, 8, 128), jnp.int32),
            pltpu.VMEM((128, emb_dim), jnp.float32),
            pltpu.VMEM((128, emb_dim), jnp.float32),
            pltpu.VMEM((128 * emb_dim,), jnp.float32),
            pltpu.VMEM((128 * emb_dim,), jnp.float32),
            pltpu.SemaphoreType.DMA,
            pltpu.SemaphoreType.DMA,
            pltpu.SemaphoreType.DMA,
        ],
        compiler_params=pltpu.CompilerParams(
            use_tc_tiling_on_sc=False,
            needs_layout_passes=False,
            disable_bounds_checks=True,
        ),
    )
    def gather(
        table_hbm, idx_hbm, out_hbm, idx_v, rows0, rows1, pan0, pan1, isem, gsem, osem
    ):
        rows = (rows0, rows1)
        pans = (pan0, pan1)
        wid = lax.axis_index("s") * info.num_cores + lax.axis_index("c")
        t0 = wid * tiles_per_w

        def idx_copy(rt):
            t = t0 + rt
            return pltpu.make_async_copy(
                idx_hbm.at[t // bb_n].at[t % bb_n], idx_v.at[rt % 2], isem
            )

        def gather_copy(rt, lin, s):
            return pltpu.make_async_copy(
                table_hbm.at[idx_v.at[rt % 2].at[lin]], rows[s], gsem
            )

        def out_copy(j, eblk, s):
            t = t0 + j // 8
            row = ((t // bb_n) * 8 + (j % 8)) * (eb_n * bb_n) + eblk * bb_n + (t % bb_n)
            return pltpu.make_async_copy(
                pans[s].at[pl.ds(eblk * 1024, 1024)], out_hbm.at[row], osem
            )

        iota16 = jnp.arange(16, dtype=jnp.int32)
        jiota = [iota16 + (jb * 16) for jb in range(8)]

        idx_copy(0).start()
        idx_copy(0).wait()
        if tiles_per_w > 1:
            idx_copy(1).start()
        gather_copy(0, 0, 0).start()

        def body(i, carry):
            for s in (0, 1):
                j = 2 * i + s
                gather_copy(0, 0, s).wait()

                @pl.when(j + 1 < jobs_per_w)
                def _():
                    rt1 = (j + 1) // 8
                    lin1 = (j + 1) % 8

                    @pl.when(lin1 == 0)
                    def _():
                        idx_copy(0).wait()

                        @pl.when(rt1 + 1 < tiles_per_w)
                        def _():
                            idx_copy(rt1 + 1).start()

                    gather_copy(rt1, lin1, 1 - s).start()

                @pl.when(j >= 2)
                def _():
                    for eblk in range(eb_n):
                        out_copy(0, eblk, s).wait()

                for e0 in range(emb_dim):
                    colv = (iota16 + e0) & (emb_dim - 1)
                    colsh = colv << 7
                    for jb in range(8):
                        v = plsc.load_gather(rows[s], [jiota[jb], colv])
                        plsc.store_scatter(pans[s], [colsh + jiota[jb]], v)

                for eblk in range(eb_n):
                    out_copy(j, eblk, s).start()
            return carry

        lax.fori_loop(0, jobs_per_w // 2, body, 0)
        for s in (0, 1):
            for eblk in range(eb_n):
                out_copy(0, eblk, s).wait()

    return gather


def kernel(x, embedding_matrix):
    b, l = x.shape
    n_emb, emb_dim = embedding_matrix.shape
    xp = x.reshape(b // 128, 128, l // 8, 8).transpose(2, 0, 3, 1)
    out = _make_gather(b, l, emb_dim)(embedding_matrix, xp)
    out5 = out.reshape(l, emb_dim // 8, b // 128, 8, 128)
    return out5.transpose(2, 4, 0, 1, 3).reshape(b, l, emb_dim)


# speedup vs baseline: 2.4353x; 1.5891x over previous
import functools

import jax
import jax.numpy as jnp
from jax import lax
from jax.experimental import pallas as pl
from jax.experimental.pallas import tpu as pltpu
from jax.experimental.pallas import tpu_sc as plsc


def _make_gather(n_rows: int, seq: int, emb_dim: int):
    info = plsc.get_sparse_core_info()
    nw = info.num_cores * info.num_subcores
    lb_n = seq // 8
    bb_n = n_rows // 128
    n_tiles = lb_n * bb_n
    tiles_per_w = n_tiles // nw
    jobs_per_w = tiles_per_w * 8
    eb_n = emb_dim // 8

    mesh = plsc.VectorSubcoreMesh(core_axis_name="c", subcore_axis_name="s")

    @functools.partial(
        pl.kernel,
        mesh=mesh,
        out_type=jax.ShapeDtypeStruct((seq * eb_n * bb_n, 8 * 128), jnp.float32),
        scratch_types=[
            pltpu.VMEM((2, 8, 128), jnp.int32),
            pltpu.VMEM((128, emb_dim), jnp.float32),
            pltpu.VMEM((128, emb_dim), jnp.float32),
            pltpu.VMEM((128 * emb_dim,), jnp.float32),
            pltpu.VMEM((128 * emb_dim,), jnp.float32),
            pltpu.SemaphoreType.DMA,
            pltpu.SemaphoreType.DMA,
            pltpu.SemaphoreType.DMA,
        ],
        compiler_params=pltpu.CompilerParams(
            use_tc_tiling_on_sc=False,
            needs_layout_passes=False,
            disable_bounds_checks=True,
        ),
    )
    def gather(
        table_hbm, idx_hbm, out_hbm, idx_v, rows0, rows1, pan0, pan1, isem, gsem, osem
    ):
        rows = (rows0, rows1)
        pans = (pan0, pan1)
        wid = lax.axis_index("s") * info.num_cores + lax.axis_index("c")
        t0 = wid * tiles_per_w

        def idx_copy(rt):
            t = t0 + rt
            return pltpu.make_async_copy(
                idx_hbm.at[t // bb_n].at[t % bb_n], idx_v.at[rt % 2], isem
            )

        def gather_copy(rt, lin, s):
            return pltpu.make_async_copy(
                table_hbm.at[idx_v.at[rt % 2].at[lin]], rows[s], gsem
            )

        def out_copy(j, eblk, s):
            t = t0 + j // 8
            row = ((t // bb_n) * 8 + (j % 8)) * (eb_n * bb_n) + eblk * bb_n + (t % bb_n)
            return pltpu.make_async_copy(
                pans[s].at[pl.ds(eblk * 1024, 1024)], out_hbm.at[row], osem
            )

        iota16 = jnp.arange(16, dtype=jnp.int32)
        jiota = [iota16 + (jb * 16) for jb in range(8)]

        idx_copy(0).start()
        idx_copy(0).wait()
        if tiles_per_w > 1:
            idx_copy(1).start()
        gather_copy(0, 0, 0).start()

        def body(i, carry):
            for s in (0, 1):
                j = 2 * i + s
                gather_copy(0, 0, s).wait()

                @pl.when(j + 1 < jobs_per_w)
                def _():
                    rt1 = (j + 1) // 8
                    lin1 = (j + 1) % 8

                    @pl.when(lin1 == 0)
                    def _():
                        idx_copy(0).wait()

                        @pl.when(rt1 + 1 < tiles_per_w)
                        def _():
                            idx_copy(rt1 + 1).start()

                    gather_copy(rt1, lin1, 1 - s).start()

                @pl.when(j >= 2)
                def _():
                    for eblk in range(eb_n):
                        out_copy(0, eblk, s).wait()

                def tbody(e0, c):
                    colv = (iota16 + e0) & (emb_dim - 1)
                    colsh = colv << 7
                    for jb in range(8):
                        v = plsc.load_gather(rows[s], [jiota[jb], colv])
                        plsc.store_scatter(pans[s], [colsh + jiota[jb]], v)
                    return c

                lax.fori_loop(0, emb_dim, tbody, 0)

                for eblk in range(eb_n):
                    out_copy(j, eblk, s).start()
            return carry

        lax.fori_loop(0, jobs_per_w // 2, body, 0)
        for s in (0, 1):
            for eblk in range(eb_n):
                out_copy(0, eblk, s).wait()

    return gather


def kernel(x, embedding_matrix):
    b, l = x.shape
    n_emb, emb_dim = embedding_matrix.shape
    xp = x.reshape(b // 128, 128, l // 8, 8).transpose(2, 0, 3, 1)
    out = _make_gather(b, l, emb_dim)(embedding_matrix, xp)
    out5 = out.reshape(l, emb_dim // 8, b // 128, 8, 128)
    return out5.transpose(2, 4, 0, 1, 3).reshape(b, l, emb_dim)


# speedup vs baseline: 2.6790x; 1.1001x over previous
import functools

import jax
import jax.numpy as jnp
from jax import lax
from jax.experimental import pallas as pl
from jax.experimental.pallas import tpu as pltpu
from jax.experimental.pallas import tpu_sc as plsc


def _make_gather(n_rows: int, seq: int, emb_dim: int):
    info = plsc.get_sparse_core_info()
    nw = info.num_cores * info.num_subcores
    lb_n = seq // 8
    bb_n = n_rows // 128
    n_tiles = lb_n * bb_n
    tiles_per_w = n_tiles // nw
    jobs_per_w = tiles_per_w * 8
    eb_n = emb_dim // 8

    mesh = plsc.VectorSubcoreMesh(core_axis_name="c", subcore_axis_name="s")

    @functools.partial(
        pl.kernel,
        mesh=mesh,
        out_type=jax.ShapeDtypeStruct((seq * eb_n * bb_n, 8 * 128), jnp.float32),
        scratch_types=[
            pltpu.VMEM((2, 8, 128), jnp.int32),
            pltpu.VMEM((128, emb_dim), jnp.float32),
            pltpu.VMEM((128, emb_dim), jnp.float32),
            pltpu.VMEM((128 * emb_dim,), jnp.float32),
            pltpu.VMEM((128 * emb_dim,), jnp.float32),
            pltpu.SemaphoreType.DMA,
            pltpu.SemaphoreType.DMA,
            pltpu.SemaphoreType.DMA,
        ],
        compiler_params=pltpu.CompilerParams(
            use_tc_tiling_on_sc=False,
            needs_layout_passes=False,
            disable_bounds_checks=True,
        ),
    )
    def gather(
        table_hbm, idx_hbm, out_hbm, idx_v, rows0, rows1, pan0, pan1, isem, gsem, osem
    ):
        rows = (rows0, rows1)
        pans = (pan0, pan1)
        wid = lax.axis_index("s") * info.num_cores + lax.axis_index("c")
        t0 = wid * tiles_per_w

        def idx_copy(rt):
            t = t0 + rt
            return pltpu.make_async_copy(
                idx_hbm.at[t // bb_n].at[t % bb_n], idx_v.at[rt % 2], isem
            )

        def gather_copy(rt, lin, s):
            return pltpu.make_async_copy(
                table_hbm.at[idx_v.at[rt % 2].at[lin]], rows[s], gsem
            )

        def out_copy(j, eblk, s):
            t = t0 + j // 8
            row = ((t // bb_n) * 8 + (j % 8)) * (eb_n * bb_n) + eblk * bb_n + (t % bb_n)
            return pltpu.make_async_copy(
                pans[s].at[pl.ds(eblk * 1024, 1024)], out_hbm.at[row], osem
            )

        iota16 = jnp.arange(16, dtype=jnp.int32)
        jiota = [iota16 + (jb * 16) for jb in range(8)]

        idx_copy(0).start()
        idx_copy(0).wait()
        if tiles_per_w > 1:
            idx_copy(1).start()
        gather_copy(0, 0, 0).start()

        def body(i, carry):
            for s in (0, 1):
                j = 2 * i + s
                gather_copy(0, 0, s).wait()

                @pl.when(j + 1 < jobs_per_w)
                def _():
                    rt1 = (j + 1) // 8
                    lin1 = (j + 1) % 8

                    @pl.when(lin1 == 0)
                    def _():
                        idx_copy(0).wait()

                        @pl.when(rt1 + 1 < tiles_per_w)
                        def _():
                            idx_copy(rt1 + 1).start()

                    gather_copy(rt1, lin1, 1 - s).start()

                @pl.when(j >= 2)
                def _():
                    for eblk in range(eb_n):
                        out_copy(0, eblk, s).wait()

                @plsc.parallel_loop(0, emb_dim, unroll=4)
                def _(e0):
                    colv = (iota16 + e0) & (emb_dim - 1)
                    colsh = colv << 7
                    for jb in range(8):
                        v = plsc.load_gather(rows[s], [jiota[jb], colv])
                        plsc.store_scatter(pans[s], [colsh + jiota[jb]], v)

                for eblk in range(eb_n):
                    out_copy(j, eblk, s).start()
            return carry

        lax.fori_loop(0, jobs_per_w // 2, body, 0)
        for s in (0, 1):
            for eblk in range(eb_n):
                out_copy(0, eblk, s).wait()

    return gather


def kernel(x, embedding_matrix):
    b, l = x.shape
    n_emb, emb_dim = embedding_matrix.shape
    xp = x.reshape(b // 128, 128, l // 8, 8).transpose(2, 0, 3, 1)
    out = _make_gather(b, l, emb_dim)(embedding_matrix, xp)
    out5 = out.reshape(l, emb_dim // 8, b // 128, 8, 128)
    return out5.transpose(2, 4, 0, 1, 3).reshape(b, l, emb_dim)


# speedup vs baseline: 2.6792x; 1.0001x over previous
import functools

import jax
import jax.numpy as jnp
from jax import lax
from jax.experimental import pallas as pl
from jax.experimental.pallas import tpu as pltpu
from jax.experimental.pallas import tpu_sc as plsc


def _make_gather(n_rows: int, seq: int, emb_dim: int):
    info = plsc.get_sparse_core_info()
    nw = info.num_cores * info.num_subcores
    lb_n = seq // 8
    bb_n = n_rows // 128
    n_tiles = lb_n * bb_n
    tiles_per_w = n_tiles // nw
    jobs_per_w = tiles_per_w * 8
    eb_n = emb_dim // 8

    mesh = plsc.VectorSubcoreMesh(core_axis_name="c", subcore_axis_name="s")

    @functools.partial(
        pl.kernel,
        mesh=mesh,
        out_type=jax.ShapeDtypeStruct((seq * eb_n * bb_n, 8 * 128), jnp.float32),
        scratch_types=[
            pltpu.VMEM((2, 8, 128), jnp.int32),
            pltpu.VMEM((128, emb_dim), jnp.float32),
            pltpu.VMEM((128, emb_dim), jnp.float32),
            pltpu.VMEM((128 * emb_dim,), jnp.float32),
            pltpu.VMEM((128 * emb_dim,), jnp.float32),
            pltpu.SemaphoreType.DMA,
            pltpu.SemaphoreType.DMA,
            pltpu.SemaphoreType.DMA,
        ],
        compiler_params=pltpu.CompilerParams(
            use_tc_tiling_on_sc=False,
            needs_layout_passes=False,
            disable_bounds_checks=True,
        ),
    )
    def gather(
        table_hbm, idx_hbm, out_hbm, idx_v, rows0, rows1, pan0, pan1, isem, gsem, osem
    ):
        rows = (rows0, rows1)
        pans = (pan0, pan1)
        wid = lax.axis_index("s") * info.num_cores + lax.axis_index("c")
        t0 = wid * tiles_per_w

        def idx_copy(rt):
            t = t0 + rt
            return pltpu.make_async_copy(
                idx_hbm.at[t // bb_n].at[t % bb_n], idx_v.at[rt % 2], isem
            )

        def gather_copy(rt, lin, s):
            return pltpu.make_async_copy(
                table_hbm.at[idx_v.at[rt % 2].at[lin]], rows[s], gsem
            )

        def out_copy(j, eblk, s):
            t = t0 + j // 8
            row = ((t // bb_n) * 8 + (j % 8)) * (eb_n * bb_n) + eblk * bb_n + (t % bb_n)
            return pltpu.make_async_copy(
                pans[s].at[pl.ds(eblk * 1024, 1024)], out_hbm.at[row], osem
            )

        iota16 = jnp.arange(16, dtype=jnp.int32)
        jiota = [iota16 + (jb * 16) for jb in range(8)]

        idx_copy(0).start()
        idx_copy(0).wait()
        if tiles_per_w > 1:
            idx_copy(1).start()
        gather_copy(0, 0, 0).start()

        def body(i, carry):
            for s in (0, 1):
                j = 2 * i + s
                gather_copy(0, 0, s).wait()

                @pl.when(j + 1 < jobs_per_w)
                def _():
                    rt1 = (j + 1) // 8
                    lin1 = (j + 1) % 8

                    @pl.when(lin1 == 0)
                    def _():
                        idx_copy(0).wait()

                        @pl.when(rt1 + 1 < tiles_per_w)
                        def _():
                            idx_copy(rt1 + 1).start()

                    gather_copy(rt1, lin1, 1 - s).start()

                @pl.when(j >= 2)
                def _():
                    for eblk in range(eb_n):
                        out_copy(0, eblk, s).wait()

                @plsc.parallel_loop(0, emb_dim, unroll=8)
                def _(e0):
                    colv = (iota16 + e0) & (emb_dim - 1)
                    colsh = colv << 7
                    for jb in range(8):
                        v = plsc.load_gather(rows[s], [jiota[jb], colv])
                        plsc.store_scatter(pans[s], [colsh + jiota[jb]], v)

                for eblk in range(eb_n):
                    out_copy(j, eblk, s).start()
            return carry

        lax.fori_loop(0, jobs_per_w // 2, body, 0)
        for s in (0, 1):
            for eblk in range(eb_n):
                out_copy(0, eblk, s).wait()

    return gather


def kernel(x, embedding_matrix):
    b, l = x.shape
    n_emb, emb_dim = embedding_matrix.shape
    xp = x.reshape(b // 128, 128, l // 8, 8).transpose(2, 0, 3, 1)
    out = _make_gather(b, l, emb_dim)(embedding_matrix, xp)
    out5 = out.reshape(l, emb_dim // 8, b // 128, 8, 128)
    return out5.transpose(2, 4, 0, 1, 3).reshape(b, l, emb_dim)
